# baseline (speedup 1.0000x reference)
import functools

import jax
import jax.numpy as jnp
from jax import lax
from jax.experimental import pallas as pl
from jax.experimental.pallas import tpu as pltpu
from jax.experimental.pallas import tpu_sc as plsc

N_NODES = 10000
N_EDGES = 320000
IN_CH = 128
HID_CH = 256
OUT_CH = 256

NC = 2
NS = 16
NW = NC * NS
L = 16
R = 320
N_PAD = NW * R
CH = 2000
G = 64
CB = HID_CH // L

_EPS = 1e-5


def _segmax_body(b_hbm, src_hbm, dst_hbm, out_hbm,
                 acc_v, dstc_v, srcc_v, csrc_v, cdloc_v, rows_v):
    w = lax.axis_index("s") * NC + lax.axis_index("c")
    lo = w * R

    neg_inf = jnp.full((L,), -jnp.inf, dtype=jnp.float32)

    @pl.loop(0, R)
    def _(r):
        for cb in range(CB):
            acc_v[r, pl.ds(cb * L, L)] = neg_inf

    iota = lax.iota(jnp.int32, L)

    def chunk_body(k, carry):
        pltpu.sync_copy(dst_hbm.at[pl.ds(k * CH, CH)], dstc_v)
        pltpu.sync_copy(src_hbm.at[pl.ds(k * CH, CH)], srcc_v)

        def group(g, off):
            d16 = dstc_v[pl.ds(g * L, L)]
            s16 = srcc_v[pl.ds(g * L, L)]
            m = (d16 >= lo) & (d16 < lo + R)
            mi = m.astype(jnp.int32)
            pos = off + plsc.cumsum(mi) - mi
            plsc.store_scatter(csrc_v, [pos], s16, mask=m)
            plsc.store_scatter(cdloc_v, [pos], d16 - lo, mask=m)
            return off + plsc.all_reduce_population_count(m)

        off = lax.fori_loop(0, CH // L, group,
                            jnp.zeros((L,), jnp.int32))
        cnt = jnp.max(off)

        for j in range(G // L):
            plsc.store_scatter(csrc_v,
                               [jnp.full((L,), cnt) + iota + (j * L)],
                               jnp.zeros((L,), jnp.int32))

        nb = (cnt + (G - 1)) // G

        def gblock(b, _):
            pltpu.sync_copy(b_hbm.at[csrc_v.at[pl.ds(b * G, G)]], rows_v)
            lim = jnp.minimum(cnt - b * G, G)

            def edge(i, _2):
                dloc = cdloc_v[pl.ds(b * G + i, L)][0]
                for cb in range(CB):
                    sl = pl.ds(cb * L, L)
                    acc_v[dloc, sl] = jnp.maximum(acc_v[dloc, sl],
                                                  rows_v[i, sl])
                return 0

            lax.fori_loop(0, lim, edge, 0)
            return 0

        lax.fori_loop(0, nb, gblock, 0)
        return carry

    lax.fori_loop(0, N_EDGES // CH, chunk_body, 0)
    pltpu.sync_copy(acc_v, out_hbm.at[pl.ds(lo, R)])


def _sc_segmax(b_tab, src, dst):
    mesh = plsc.VectorSubcoreMesh(core_axis_name="c", subcore_axis_name="s",
                                  num_cores=NC, num_subcores=NS)
    f = pl.kernel(
        _segmax_body,
        out_type=jax.ShapeDtypeStruct((N_PAD, HID_CH), jnp.float32),
        mesh=mesh,
        compiler_params=pltpu.CompilerParams(needs_layout_passes=False),
        scratch_types=[
            pltpu.VMEM((R, HID_CH), jnp.float32),
            pltpu.VMEM((CH,), jnp.int32),
            pltpu.VMEM((CH,), jnp.int32),
            pltpu.VMEM((CH + G + L,), jnp.int32),
            pltpu.VMEM((CH + G + L,), jnp.int32),
            pltpu.VMEM((G, HID_CH), jnp.float32),
        ],
    )
    return f(b_tab, src, dst)


def _mm(a, w):
    return lax.dot_general(a, w, (((1,), (0,)), ((), ())),
                           precision=lax.Precision.HIGHEST,
                           preferred_element_type=jnp.float32)


RB = 1000
NRB = N_NODES // RB


def _prep1_body(x_ref, w_ref, b_ref, a_ref, bt_ref):
    wfull = w_ref[...]
    wt = wfull[:IN_CH]
    wb = wfull[IN_CH:]
    xv = x_ref[...]
    a_ref[...] = _mm(xv, wt - wb) + b_ref[...]
    bt_ref[...] = _mm(xv, wb)


def _tc_prep1(x, W1, b1):
    return pl.pallas_call(
        _prep1_body,
        grid=(NRB,),
        in_specs=[pl.BlockSpec((RB, IN_CH), lambda i: (i, 0)),
                  pl.BlockSpec((2 * IN_CH, HID_CH), lambda i: (0, 0)),
                  pl.BlockSpec((1, HID_CH), lambda i: (0, 0))],
        out_specs=[pl.BlockSpec((RB, HID_CH), lambda i: (i, 0)),
                   pl.BlockSpec((RB, HID_CH), lambda i: (i, 0))],
        out_shape=[jax.ShapeDtypeStruct((N_NODES, HID_CH), jnp.float32),
                   jax.ShapeDtypeStruct((N_NODES, HID_CH), jnp.float32)],
    )(x, W1, b1.reshape(1, HID_CH))


def _stats_body(a_ref, m_ref, hp_ref, s_ref, q_ref):
    @pl.when(pl.program_id(0) == 0)
    def _():
        s_ref[...] = jnp.zeros((1, 128), jnp.float32)
        q_ref[...] = jnp.zeros((1, 128), jnp.float32)

    mv = m_ref[...]
    hp = jnp.where(jnp.isfinite(mv), a_ref[...] + mv, 0.0)
    hp_ref[...] = hp
    s_ref[...] += jnp.full((1, 128), jnp.sum(hp))
    q_ref[...] += jnp.full((1, 128), jnp.sum(hp * hp))


def _tc_stats(A1, M1):
    return pl.pallas_call(
        _stats_body,
        grid=(NRB,),
        in_specs=[pl.BlockSpec((RB, HID_CH), lambda i: (i, 0)),
                  pl.BlockSpec((RB, HID_CH), lambda i: (i, 0))],
        out_specs=[pl.BlockSpec((RB, HID_CH), lambda i: (i, 0)),
                   pl.BlockSpec((1, 128), lambda i: (0, 0)),
                   pl.BlockSpec((1, 128), lambda i: (0, 0))],
        out_shape=[jax.ShapeDtypeStruct((N_NODES, HID_CH), jnp.float32),
                   jax.ShapeDtypeStruct((1, 128), jnp.float32),
                   jax.ShapeDtypeStruct((1, 128), jnp.float32)],
    )(A1, M1)


def _norm_prep_body(hp_ref, s_ref, q_ref, lnw_ref, lnb_ref, pa_ref,
                    w_ref, b_ref, a_ref, bt_ref):
    n_elem = float(N_NODES * HID_CH)
    mean = jnp.max(s_ref[...]) / n_elem
    var = jnp.max(q_ref[...]) / n_elem - mean * mean
    inv = lax.rsqrt(var + _EPS)
    h = (hp_ref[...] - mean) * inv * lnw_ref[...] + lnb_ref[...]
    pa = jnp.max(pa_ref[...])
    h = jnp.where(h >= 0, h, pa * h)
    wfull = w_ref[...]
    wt = wfull[:HID_CH]
    wb = wfull[HID_CH:]
    a_ref[...] = _mm(h, wt - wb) + b_ref[...]
    bt_ref[...] = _mm(h, wb)


def _tc_norm_prep(hp, s, q, ln_w, ln_b, prelu_a, W2, b2):
    return pl.pallas_call(
        _norm_prep_body,
        grid=(NRB,),
        in_specs=[pl.BlockSpec((RB, HID_CH), lambda i: (i, 0)),
                  pl.BlockSpec((1, 128), lambda i: (0, 0)),
                  pl.BlockSpec((1, 128), lambda i: (0, 0)),
                  pl.BlockSpec((1, HID_CH), lambda i: (0, 0)),
                  pl.BlockSpec((1, HID_CH), lambda i: (0, 0)),
                  pl.BlockSpec((1, 1), lambda i: (0, 0)),
                  pl.BlockSpec((2 * HID_CH, OUT_CH), lambda i: (0, 0)),
                  pl.BlockSpec((1, OUT_CH), lambda i: (0, 0))],
        out_specs=[pl.BlockSpec((RB, OUT_CH), lambda i: (i, 0)),
                   pl.BlockSpec((RB, OUT_CH), lambda i: (i, 0))],
        out_shape=[jax.ShapeDtypeStruct((N_NODES, OUT_CH), jnp.float32),
                   jax.ShapeDtypeStruct((N_NODES, OUT_CH), jnp.float32)],
    )(hp, s, q, ln_w.reshape(1, HID_CH), ln_b.reshape(1, HID_CH),
      prelu_a.reshape(1, 1), W2, b2.reshape(1, OUT_CH))


def _final_body(a_ref, m_ref, o_ref):
    mv = m_ref[...]
    o_ref[...] = jnp.where(jnp.isfinite(mv), a_ref[...] + mv, 0.0)


def _tc_final(A2, M2):
    return pl.pallas_call(
        _final_body,
        grid=(NRB,),
        in_specs=[pl.BlockSpec((RB, OUT_CH), lambda i: (i, 0)),
                  pl.BlockSpec((RB, OUT_CH), lambda i: (i, 0))],
        out_specs=pl.BlockSpec((RB, OUT_CH), lambda i: (i, 0)),
        out_shape=jax.ShapeDtypeStruct((N_NODES, OUT_CH), jnp.float32),
    )(A2, M2)


def kernel(x, edge_index, W1, b1, W2, b2, ln_w, ln_b, prelu_a):
    ei = edge_index.astype(jnp.int32)
    src = ei[0]
    dst = ei[1]
    A1, B1 = _tc_prep1(x, W1, b1)
    M1 = _sc_segmax(B1, src, dst)[:N_NODES]
    hp, s, q = _tc_stats(A1, M1)
    A2, B2 = _tc_norm_prep(hp, s, q, ln_w, ln_b, prelu_a, W2, b2)
    M2 = _sc_segmax(B2, src, dst)[:N_NODES]
    return _tc_final(A2, M2)


# speedup vs baseline: 6.5465x; 6.5465x over previous
import functools

import jax
import jax.numpy as jnp
from jax import lax
from jax.experimental import pallas as pl
from jax.experimental.pallas import tpu as pltpu
from jax.experimental.pallas import tpu_sc as plsc

N_NODES = 10000
N_EDGES = 320000
IN_CH = 128
HID_CH = 256
OUT_CH = 256

NC = 2
NS = 16
NW = NC * NS
L = 16
R = 320
N_PAD = NW * R
CH = 8000
NCH = N_EDGES // CH
PCH = 2048
G = 64
CB = HID_CH // L
DUMMY = R << 14
PAIR_W = N_EDGES + CH + L * NCH + 192

_EPS = 1e-5

def _al(v, m=8):
    return pl.multiple_of(v, m)


_SC_MESH = dict(core_axis_name="c", subcore_axis_name="s",
                num_cores=NC, num_subcores=NS)


def _bin_body(src_hbm, dst_hbm, pairs_hbm, counts_hbm,
              dstc0, dstc1, srcc0, srcc1, cpair_v, stage_v, sem0, sem1):
    w = lax.axis_index("s") * NC + lax.axis_index("c")
    lo = w * R
    iota = lax.iota(jnp.int32, L)

    def start_chunk(k, s):
        sem = sem0 if s == 0 else sem1
        dc = dstc0 if s == 0 else dstc1
        sc = srcc0 if s == 0 else srcc1
        pltpu.async_copy(dst_hbm.at[pl.ds(_al(k * CH, 16), CH)], dc, sem)
        pltpu.async_copy(src_hbm.at[pl.ds(_al(k * CH, 16), CH)], sc, sem)

    def wait_chunk(k, s):
        sem = sem0 if s == 0 else sem1
        dc = dstc0 if s == 0 else dstc1
        sc = srcc0 if s == 0 else srcc1
        pltpu.make_async_copy(dst_hbm.at[pl.ds(_al(k * CH, 16), CH)],
                              dc, sem).wait()
        pltpu.make_async_copy(src_hbm.at[pl.ds(_al(k * CH, 16), CH)],
                              sc, sem).wait()

    start_chunk(0, 0)

    def do_chunk(k, s, goff):
        @pl.when(k + 1 < NCH)
        def _():
            start_chunk(k + 1, 1 - s)

        wait_chunk(k, s)

        dc = dstc0 if s == 0 else dstc1
        sc = srcc0 if s == 0 else srcc1

        def group(g, off):
            d16 = dc[pl.ds(_al(g * L, 16), L)]
            s16 = sc[pl.ds(_al(g * L, 16), L)]
            m = (d16 >= lo) & (d16 < lo + R)
            mi = m.astype(jnp.int32)
            pos = off + plsc.cumsum(mi) - mi
            pair = s16 | ((d16 - lo) << 14)
            plsc.store_scatter(cpair_v, [pos], pair, mask=m)
            return off + plsc.all_reduce_population_count(m)

        off = lax.fori_loop(0, CH // L, group, jnp.zeros((L,), jnp.int32))
        cnt = jnp.max(off)
        plsc.store_scatter(cpair_v, [jnp.full((L,), cnt) + iota],
                           jnp.full((L,), DUMMY, jnp.int32))
        cnt16 = jnp.bitwise_and(cnt + (L - 1), -L)
        pltpu.sync_copy(cpair_v.at[pl.ds(0, CH + L)],
                        pairs_hbm.at[pl.ds(_al(w * PAIR_W + goff, 16), CH + L)])
        return goff + cnt16

    def chunk_pair(j, goff):
        goff = do_chunk(2 * j, 0, goff)
        goff = do_chunk(2 * j + 1, 1, goff)
        return goff

    goff = lax.fori_loop(0, NCH // 2, chunk_pair, 0)

    for jj in range(G // L):
        cpair_v[pl.ds(jj * L, L)] = jnp.full((L,), DUMMY, jnp.int32)
    pltpu.sync_copy(cpair_v.at[pl.ds(0, G)],
                    pairs_hbm.at[pl.ds(_al(w * PAIR_W + goff, 16), G)])
    t64 = jnp.bitwise_and(goff + (G - 1), -G)
    stage_v[...] = jnp.full((L,), t64, jnp.int32)
    pltpu.sync_copy(stage_v, counts_hbm.at[pl.ds(_al(w * L, 16), L)])


def _sc_bin(src, dst):
    f = pl.kernel(
        _bin_body,
        out_type=[jax.ShapeDtypeStruct((NW * PAIR_W,), jnp.int32),
                  jax.ShapeDtypeStruct((NW * L,), jnp.int32)],
        mesh=plsc.VectorSubcoreMesh(**_SC_MESH),
        compiler_params=pltpu.CompilerParams(needs_layout_passes=False),
        scratch_types=[
            pltpu.VMEM((CH,), jnp.int32),
            pltpu.VMEM((CH,), jnp.int32),
            pltpu.VMEM((CH,), jnp.int32),
            pltpu.VMEM((CH,), jnp.int32),
            pltpu.VMEM((CH + 2 * L,), jnp.int32),
            pltpu.VMEM((L,), jnp.int32),
            pltpu.SemaphoreType.DMA,
            pltpu.SemaphoreType.DMA,
        ],
    )
    return f(src, dst)


def _segmax_body(b_hbm, pairs_hbm, counts_hbm, out_hbm,
                 acc_v, pairc0, pairc1, csrc_v, cdloc_v, rows0, rows1,
                 stage_v, psem0, psem1, rsem0, rsem1):
    w = lax.axis_index("s") * NC + lax.axis_index("c")
    lo = w * R

    neg_inf = jnp.full((L,), -jnp.inf, dtype=jnp.float32)

    @pl.loop(0, R + 1)
    def _(r):
        for cb in range(CB):
            acc_v[r, pl.ds(cb * L, L)] = neg_inf

    pltpu.sync_copy(counts_hbm.at[pl.ds(_al(w * L, 16), L)], stage_v)
    t64 = stage_v[...][0]
    np_ = (t64 + (PCH - 1)) // PCH

    def start_pairs(p, s):
        sem = psem0 if s == 0 else psem1
        pc = pairc0 if s == 0 else pairc1
        pltpu.async_copy(pairs_hbm.at[pl.ds(_al(w * PAIR_W + p * PCH, 16), PCH)],
                         pc, sem)

    def wait_pairs(p, s):
        sem = psem0 if s == 0 else psem1
        pc = pairc0 if s == 0 else pairc1
        pltpu.make_async_copy(pairs_hbm.at[pl.ds(_al(w * PAIR_W + p * PCH, 16), PCH)],
                              pc, sem).wait()

    def start_rows(b, rs):
        sem = rsem0 if rs == 0 else rsem1
        rv = rows0 if rs == 0 else rows1
        pltpu.async_copy(b_hbm.at[csrc_v.at[pl.ds(_al(b * G, 16), G)]],
                         rv, sem)

    def wait_rows(b, rs):
        sem = rsem0 if rs == 0 else rsem1
        rv = rows0 if rs == 0 else rows1
        pltpu.make_async_copy(b_hbm.at[csrc_v.at[pl.ds(_al(b * G, 16), G)]],
                              rv, sem).wait()

    def rmw(b, rs):
        rv = rows0 if rs == 0 else rows1

        @pl.loop(0, 4)
        def _(q):
            dv = cdloc_v[pl.ds(_al(b * G + q * L, 16), L)]
            for i in range(L):
                d = dv[i]
                for cb in range(CB):
                    sl = pl.ds(cb * L, L)
                    acc_v[d, sl] = jnp.maximum(acc_v[d, sl],
                                               rv[q * L + i, sl])

    start_pairs(0, 0)

    def do_pchunk(p, s):
        @pl.when(p + 1 < np_)
        def _():
            start_pairs(p + 1, 1 - s)

        wait_pairs(p, s)
        pc = pairc0 if s == 0 else pairc1
        nb = jnp.minimum(t64 - p * PCH, PCH) // G

        @pl.loop(0, nb * (G // L))
        def _(g):
            pr = pc[pl.ds(_al(g * L, 16), L)]
            csrc_v[pl.ds(_al(g * L, 16), L)] = pr & (16384 - 1)
            cdloc_v[pl.ds(_al(g * L, 16), L)] = pr >> 14

        start_rows(0, 0)

        def gpair(jb, _):
            b = 2 * jb

            @pl.when(b + 1 < nb)
            def _():
                start_rows(b + 1, 1)

            wait_rows(b, 0)
            rmw(b, 0)

            @pl.when(b + 2 < nb)
            def _():
                start_rows(b + 2, 0)

            @pl.when(b + 1 < nb)
            def _():
                wait_rows(b + 1, 1)
                rmw(b + 1, 1)

            return 0

        lax.fori_loop(0, (nb + 1) // 2, gpair, 0)

    def pchunk_pair(jp, _):
        do_pchunk(2 * jp, 0)

        @pl.when(2 * jp + 1 < np_)
        def _():
            do_pchunk(2 * jp + 1, 1)

        return 0

    lax.fori_loop(0, (np_ + 1) // 2, pchunk_pair, 0)
    pltpu.sync_copy(acc_v.at[pl.ds(0, R)], out_hbm.at[pl.ds(lo, R)])


def _sc_segmax(b_tab, pairs, counts):
    f = pl.kernel(
        _segmax_body,
        out_type=jax.ShapeDtypeStruct((N_PAD, HID_CH), jnp.float32),
        mesh=plsc.VectorSubcoreMesh(**_SC_MESH),
        compiler_params=pltpu.CompilerParams(needs_layout_passes=False),
        scratch_types=[
            pltpu.VMEM((R + 1, HID_CH), jnp.float32),
            pltpu.VMEM((PCH,), jnp.int32),
            pltpu.VMEM((PCH,), jnp.int32),
            pltpu.VMEM((PCH,), jnp.int32),
            pltpu.VMEM((PCH,), jnp.int32),
            pltpu.VMEM((G, HID_CH), jnp.float32),
            pltpu.VMEM((G, HID_CH), jnp.float32),
            pltpu.VMEM((L,), jnp.int32),
            pltpu.SemaphoreType.DMA,
            pltpu.SemaphoreType.DMA,
            pltpu.SemaphoreType.DMA,
            pltpu.SemaphoreType.DMA,
        ],
    )
    return f(b_tab, pairs, counts)


def _mm(a, w):
    return lax.dot_general(a, w, (((1,), (0,)), ((), ())),
                           precision=lax.Precision.HIGHEST,
                           preferred_element_type=jnp.float32)


RB = 1000
NRB = N_NODES // RB


def _prep1_body(x_ref, w_ref, b_ref, a_ref, bt_ref):
    wfull = w_ref[...]
    wt = wfull[:IN_CH]
    wb = wfull[IN_CH:]
    xv = x_ref[...]
    a_ref[...] = _mm(xv, wt - wb) + b_ref[...]
    bt_ref[...] = _mm(xv, wb)


def _tc_prep1(x, W1, b1):
    return pl.pallas_call(
        _prep1_body,
        grid=(NRB,),
        in_specs=[pl.BlockSpec((RB, IN_CH), lambda i: (i, 0)),
                  pl.BlockSpec((2 * IN_CH, HID_CH), lambda i: (0, 0)),
                  pl.BlockSpec((1, HID_CH), lambda i: (0, 0))],
        out_specs=[pl.BlockSpec((RB, HID_CH), lambda i: (i, 0)),
                   pl.BlockSpec((RB, HID_CH), lambda i: (i, 0))],
        out_shape=[jax.ShapeDtypeStruct((N_NODES, HID_CH), jnp.float32),
                   jax.ShapeDtypeStruct((N_NODES, HID_CH), jnp.float32)],
    )(x, W1, b1.reshape(1, HID_CH))


def _stats_body(a_ref, m_ref, hp_ref, s_ref, q_ref):
    @pl.when(pl.program_id(0) == 0)
    def _():
        s_ref[...] = jnp.zeros((1, 128), jnp.float32)
        q_ref[...] = jnp.zeros((1, 128), jnp.float32)

    mv = m_ref[...]
    hp = jnp.where(jnp.isfinite(mv), a_ref[...] + mv, 0.0)
    hp_ref[...] = hp
    s_ref[...] += jnp.full((1, 128), jnp.sum(hp))
    q_ref[...] += jnp.full((1, 128), jnp.sum(hp * hp))


def _tc_stats(A1, M1):
    return pl.pallas_call(
        _stats_body,
        grid=(NRB,),
        in_specs=[pl.BlockSpec((RB, HID_CH), lambda i: (i, 0)),
                  pl.BlockSpec((RB, HID_CH), lambda i: (i, 0))],
        out_specs=[pl.BlockSpec((RB, HID_CH), lambda i: (i, 0)),
                   pl.BlockSpec((1, 128), lambda i: (0, 0)),
                   pl.BlockSpec((1, 128), lambda i: (0, 0))],
        out_shape=[jax.ShapeDtypeStruct((N_NODES, HID_CH), jnp.float32),
                   jax.ShapeDtypeStruct((1, 128), jnp.float32),
                   jax.ShapeDtypeStruct((1, 128), jnp.float32)],
    )(A1, M1)


def _norm_prep_body(hp_ref, s_ref, q_ref, lnw_ref, lnb_ref, pa_ref,
                    w_ref, b_ref, a_ref, bt_ref):
    n_elem = float(N_NODES * HID_CH)
    mean = jnp.max(s_ref[...]) / n_elem
    var = jnp.max(q_ref[...]) / n_elem - mean * mean
    inv = lax.rsqrt(var + _EPS)
    h = (hp_ref[...] - mean) * inv * lnw_ref[...] + lnb_ref[...]
    pa = jnp.max(pa_ref[...])
    h = jnp.where(h >= 0, h, pa * h)
    wfull = w_ref[...]
    wt = wfull[:HID_CH]
    wb = wfull[HID_CH:]
    a_ref[...] = _mm(h, wt - wb) + b_ref[...]
    bt_ref[...] = _mm(h, wb)


def _tc_norm_prep(hp, s, q, ln_w, ln_b, prelu_a, W2, b2):
    return pl.pallas_call(
        _norm_prep_body,
        grid=(NRB,),
        in_specs=[pl.BlockSpec((RB, HID_CH), lambda i: (i, 0)),
                  pl.BlockSpec((1, 128), lambda i: (0, 0)),
                  pl.BlockSpec((1, 128), lambda i: (0, 0)),
                  pl.BlockSpec((1, HID_CH), lambda i: (0, 0)),
                  pl.BlockSpec((1, HID_CH), lambda i: (0, 0)),
                  pl.BlockSpec((1, 1), lambda i: (0, 0)),
                  pl.BlockSpec((2 * HID_CH, OUT_CH), lambda i: (0, 0)),
                  pl.BlockSpec((1, OUT_CH), lambda i: (0, 0))],
        out_specs=[pl.BlockSpec((RB, OUT_CH), lambda i: (i, 0)),
                   pl.BlockSpec((RB, OUT_CH), lambda i: (i, 0))],
        out_shape=[jax.ShapeDtypeStruct((N_NODES, OUT_CH), jnp.float32),
                   jax.ShapeDtypeStruct((N_NODES, OUT_CH), jnp.float32)],
    )(hp, s, q, ln_w.reshape(1, HID_CH), ln_b.reshape(1, HID_CH),
      prelu_a.reshape(1, 1), W2, b2.reshape(1, OUT_CH))


def _final_body(a_ref, m_ref, o_ref):
    mv = m_ref[...]
    o_ref[...] = jnp.where(jnp.isfinite(mv), a_ref[...] + mv, 0.0)


def _tc_final(A2, M2):
    return pl.pallas_call(
        _final_body,
        grid=(NRB,),
        in_specs=[pl.BlockSpec((RB, OUT_CH), lambda i: (i, 0)),
                  pl.BlockSpec((RB, OUT_CH), lambda i: (i, 0))],
        out_specs=pl.BlockSpec((RB, OUT_CH), lambda i: (i, 0)),
        out_shape=jax.ShapeDtypeStruct((N_NODES, OUT_CH), jnp.float32),
    )(A2, M2)


def kernel(x, edge_index, W1, b1, W2, b2, ln_w, ln_b, prelu_a):
    ei = edge_index.astype(jnp.int32)
    src = ei[0]
    dst = ei[1]
    pairs, counts = _sc_bin(src, dst)
    A1, B1 = _tc_prep1(x, W1, b1)
    M1 = _sc_segmax(B1, pairs, counts)[:N_NODES]
    hp, s, q = _tc_stats(A1, M1)
    A2, B2 = _tc_norm_prep(hp, s, q, ln_w, ln_b, prelu_a, W2, b2)
    M2 = _sc_segmax(B2, pairs, counts)[:N_NODES]
    return _tc_final(A2, M2)


# speedup vs baseline: 8.3314x; 1.2726x over previous
import functools

import jax
import jax.numpy as jnp
from jax import lax
from jax.experimental import pallas as pl
from jax.experimental.pallas import tpu as pltpu
from jax.experimental.pallas import tpu_sc as plsc

N_NODES = 10000
N_EDGES = 320000
IN_CH = 128
HID_CH = 256
OUT_CH = 256

NC = 2
NS = 16
NW = NC * NS
L = 16
R = 320
N_PAD = NW * R
CH = 8000
NCH = N_EDGES // CH
PCH = 2048
G = 64
CB = HID_CH // L
L2 = 32
CB2 = HID_CH // L2
NEG_INF_2X = -8388736
DUMMY = R << 14
PAIR_W = N_EDGES + CH + L * NCH + 192

_EPS = 1e-5

def _al(v, m=8):
    return pl.multiple_of(v, m)


_SC_MESH = dict(core_axis_name="c", subcore_axis_name="s",
                num_cores=NC, num_subcores=NS)


def _bin_body(src_hbm, dst_hbm, pairs_hbm, counts_hbm,
              dstc0, dstc1, srcc0, srcc1, cpair_v, stage_v, sem0, sem1):
    w = lax.axis_index("s") * NC + lax.axis_index("c")
    lo = w * R
    iota = lax.iota(jnp.int32, L)

    def start_chunk(k, s):
        sem = sem0 if s == 0 else sem1
        dc = dstc0 if s == 0 else dstc1
        sc = srcc0 if s == 0 else srcc1
        pltpu.async_copy(dst_hbm.at[pl.ds(_al(k * CH, 16), CH)], dc, sem)
        pltpu.async_copy(src_hbm.at[pl.ds(_al(k * CH, 16), CH)], sc, sem)

    def wait_chunk(k, s):
        sem = sem0 if s == 0 else sem1
        dc = dstc0 if s == 0 else dstc1
        sc = srcc0 if s == 0 else srcc1
        pltpu.make_async_copy(dst_hbm.at[pl.ds(_al(k * CH, 16), CH)],
                              dc, sem).wait()
        pltpu.make_async_copy(src_hbm.at[pl.ds(_al(k * CH, 16), CH)],
                              sc, sem).wait()

    start_chunk(0, 0)

    def do_chunk(k, s, goff):
        @pl.when(k + 1 < NCH)
        def _():
            start_chunk(k + 1, 1 - s)

        wait_chunk(k, s)

        dc = dstc0 if s == 0 else dstc1
        sc = srcc0 if s == 0 else srcc1

        def group(g, off):
            d16 = dc[pl.ds(_al(g * L, 16), L)]
            s16 = sc[pl.ds(_al(g * L, 16), L)]
            m = (d16 >= lo) & (d16 < lo + R)
            mi = m.astype(jnp.int32)
            pos = off + plsc.cumsum(mi) - mi
            pair = s16 | ((d16 - lo) << 14)
            plsc.store_scatter(cpair_v, [pos], pair, mask=m)
            return off + plsc.all_reduce_population_count(m)

        off = lax.fori_loop(0, CH // L, group, jnp.zeros((L,), jnp.int32))
        cnt = jnp.max(off)
        plsc.store_scatter(cpair_v, [jnp.full((L,), cnt) + iota],
                           jnp.full((L,), DUMMY, jnp.int32))
        cnt16 = jnp.bitwise_and(cnt + (L - 1), -L)
        pltpu.sync_copy(cpair_v.at[pl.ds(0, CH + L)],
                        pairs_hbm.at[pl.ds(_al(w * PAIR_W + goff, 16), CH + L)])
        return goff + cnt16

    def chunk_pair(j, goff):
        goff = do_chunk(2 * j, 0, goff)
        goff = do_chunk(2 * j + 1, 1, goff)
        return goff

    goff = lax.fori_loop(0, NCH // 2, chunk_pair, 0)

    for jj in range(G // L):
        cpair_v[pl.ds(jj * L, L)] = jnp.full((L,), DUMMY, jnp.int32)
    pltpu.sync_copy(cpair_v.at[pl.ds(0, G)],
                    pairs_hbm.at[pl.ds(_al(w * PAIR_W + goff, 16), G)])
    t64 = jnp.bitwise_and(goff + (G - 1), -G)
    stage_v[...] = jnp.full((L,), t64, jnp.int32)
    pltpu.sync_copy(stage_v, counts_hbm.at[pl.ds(_al(w * L, 16), L)])


def _sc_bin(src, dst):
    f = pl.kernel(
        _bin_body,
        out_type=[jax.ShapeDtypeStruct((NW * PAIR_W,), jnp.int32),
                  jax.ShapeDtypeStruct((NW * L,), jnp.int32)],
        mesh=plsc.VectorSubcoreMesh(**_SC_MESH),
        compiler_params=pltpu.CompilerParams(needs_layout_passes=False),
        scratch_types=[
            pltpu.VMEM((CH,), jnp.int32),
            pltpu.VMEM((CH,), jnp.int32),
            pltpu.VMEM((CH,), jnp.int32),
            pltpu.VMEM((CH,), jnp.int32),
            pltpu.VMEM((CH + 2 * L,), jnp.int32),
            pltpu.VMEM((L,), jnp.int32),
            pltpu.SemaphoreType.DMA,
            pltpu.SemaphoreType.DMA,
        ],
    )
    return f(src, dst)


def _segmax_body(b_hbm, pairs_hbm, counts_hbm, out_hbm,
                 acc_v, pairc0, pairc1, csrc_v, cdloc_v, rows0, rows1,
                 stage_v, psem0, psem1, rsem0, rsem1):
    w = lax.axis_index("s") * NC + lax.axis_index("c")
    lo = w * R

    neg_inf = jnp.full((L,), NEG_INF_2X, dtype=jnp.int32)

    @pl.loop(0, R + 1)
    def _(r):
        for cb in range(CB2):
            acc_v[r, pl.ds(cb * L, L)] = neg_inf

    pltpu.sync_copy(counts_hbm.at[pl.ds(_al(w * L, 16), L)], stage_v)
    t64 = stage_v[...][0]
    np_ = (t64 + (PCH - 1)) // PCH

    def start_pairs(p, s):
        sem = psem0 if s == 0 else psem1
        pc = pairc0 if s == 0 else pairc1
        pltpu.async_copy(pairs_hbm.at[pl.ds(_al(w * PAIR_W + p * PCH, 16), PCH)],
                         pc, sem)

    def wait_pairs(p, s):
        sem = psem0 if s == 0 else psem1
        pc = pairc0 if s == 0 else pairc1
        pltpu.make_async_copy(pairs_hbm.at[pl.ds(_al(w * PAIR_W + p * PCH, 16), PCH)],
                              pc, sem).wait()

    def start_rows(b, rs):
        sem = rsem0 if rs == 0 else rsem1
        rv = rows0 if rs == 0 else rows1
        pltpu.async_copy(b_hbm.at[csrc_v.at[pl.ds(_al(b * G, 16), G)]],
                         rv, sem)

    def wait_rows(b, rs):
        sem = rsem0 if rs == 0 else rsem1
        rv = rows0 if rs == 0 else rows1
        pltpu.make_async_copy(b_hbm.at[csrc_v.at[pl.ds(_al(b * G, 16), G)]],
                              rv, sem).wait()

    def rmw(b, rs):
        rv = rows0 if rs == 0 else rows1

        @pl.loop(0, 4)
        def _(q):
            dv = cdloc_v[pl.ds(_al(b * G + q * L, 16), L)]
            for i in range(L):
                d = dv[i]
                for cb in range(CB2):
                    sl = pl.ds(cb * L, L)
                    val = plsc.bitcast(rv[q * L + i, sl], jnp.bfloat16)
                    cur = plsc.bitcast(acc_v[d, sl], jnp.bfloat16)
                    acc_v[d, sl] = plsc.bitcast(jnp.maximum(cur, val),
                                                jnp.int32)

    start_pairs(0, 0)

    def do_pchunk(p, s):
        @pl.when(p + 1 < np_)
        def _():
            start_pairs(p + 1, 1 - s)

        wait_pairs(p, s)
        pc = pairc0 if s == 0 else pairc1
        nb = jnp.minimum(t64 - p * PCH, PCH) // G

        @pl.loop(0, nb * (G // L))
        def _(g):
            pr = pc[pl.ds(_al(g * L, 16), L)]
            csrc_v[pl.ds(_al(g * L, 16), L)] = pr & (16384 - 1)
            cdloc_v[pl.ds(_al(g * L, 16), L)] = pr >> 14

        start_rows(0, 0)

        def gpair(jb, _):
            b = 2 * jb

            @pl.when(b + 1 < nb)
            def _():
                start_rows(b + 1, 1)

            wait_rows(b, 0)
            rmw(b, 0)

            @pl.when(b + 2 < nb)
            def _():
                start_rows(b + 2, 0)

            @pl.when(b + 1 < nb)
            def _():
                wait_rows(b + 1, 1)
                rmw(b + 1, 1)

            return 0

        lax.fori_loop(0, (nb + 1) // 2, gpair, 0)

    def pchunk_pair(jp, _):
        do_pchunk(2 * jp, 0)

        @pl.when(2 * jp + 1 < np_)
        def _():
            do_pchunk(2 * jp + 1, 1)

        return 0

    lax.fori_loop(0, (np_ + 1) // 2, pchunk_pair, 0)
    pltpu.sync_copy(acc_v.at[pl.ds(0, R)], out_hbm.at[pl.ds(lo, R)])


def _sc_segmax(b_tab, pairs, counts):
    b_i32 = lax.bitcast_convert_type(
        b_tab.reshape(N_NODES, HID_CH // 2, 2), jnp.int32)
    f = pl.kernel(
        _segmax_body,
        out_type=jax.ShapeDtypeStruct((N_PAD, HID_CH // 2), jnp.int32),
        mesh=plsc.VectorSubcoreMesh(**_SC_MESH),
        compiler_params=pltpu.CompilerParams(needs_layout_passes=False),
        scratch_types=[
            pltpu.VMEM((R + 1, HID_CH // 2), jnp.int32),
            pltpu.VMEM((PCH,), jnp.int32),
            pltpu.VMEM((PCH,), jnp.int32),
            pltpu.VMEM((PCH,), jnp.int32),
            pltpu.VMEM((PCH,), jnp.int32),
            pltpu.VMEM((G, HID_CH // 2), jnp.int32),
            pltpu.VMEM((G, HID_CH // 2), jnp.int32),
            pltpu.VMEM((L,), jnp.int32),
            pltpu.SemaphoreType.DMA,
            pltpu.SemaphoreType.DMA,
            pltpu.SemaphoreType.DMA,
            pltpu.SemaphoreType.DMA,
        ],
    )
    m_i32 = f(b_i32, pairs, counts)
    return lax.bitcast_convert_type(m_i32, jnp.bfloat16).reshape(
        N_PAD, HID_CH)


def _mm(a, w):
    return lax.dot_general(a, w, (((1,), (0,)), ((), ())),
                           precision=lax.Precision.HIGHEST,
                           preferred_element_type=jnp.float32)


RB = 1000
NRB = N_NODES // RB


def _prep1_body(x_ref, w_ref, b_ref, a_ref, bt_ref):
    wfull = w_ref[...]
    wt = wfull[:IN_CH]
    wb = wfull[IN_CH:]
    xv = x_ref[...]
    a_ref[...] = _mm(xv, wt - wb) + b_ref[...]
    bt_ref[...] = _mm(xv, wb).astype(jnp.bfloat16)


def _tc_prep1(x, W1, b1):
    return pl.pallas_call(
        _prep1_body,
        grid=(NRB,),
        in_specs=[pl.BlockSpec((RB, IN_CH), lambda i: (i, 0)),
                  pl.BlockSpec((2 * IN_CH, HID_CH), lambda i: (0, 0)),
                  pl.BlockSpec((1, HID_CH), lambda i: (0, 0))],
        out_specs=[pl.BlockSpec((RB, HID_CH), lambda i: (i, 0)),
                   pl.BlockSpec((RB, HID_CH), lambda i: (i, 0))],
        out_shape=[jax.ShapeDtypeStruct((N_NODES, HID_CH), jnp.float32),
                   jax.ShapeDtypeStruct((N_NODES, HID_CH), jnp.bfloat16)],
    )(x, W1, b1.reshape(1, HID_CH))


def _stats_body(a_ref, m_ref, hp_ref, s_ref, q_ref):
    @pl.when(pl.program_id(0) == 0)
    def _():
        s_ref[...] = jnp.zeros((1, 128), jnp.float32)
        q_ref[...] = jnp.zeros((1, 128), jnp.float32)

    mv = m_ref[...].astype(jnp.float32)
    hp = jnp.where(jnp.isfinite(mv), a_ref[...] + mv, 0.0)
    hp_ref[...] = hp
    s_ref[...] += jnp.full((1, 128), jnp.sum(hp))
    q_ref[...] += jnp.full((1, 128), jnp.sum(hp * hp))


def _tc_stats(A1, M1):
    return pl.pallas_call(
        _stats_body,
        grid=(NRB,),
        in_specs=[pl.BlockSpec((RB, HID_CH), lambda i: (i, 0)),
                  pl.BlockSpec((RB, HID_CH), lambda i: (i, 0))],
        out_specs=[pl.BlockSpec((RB, HID_CH), lambda i: (i, 0)),
                   pl.BlockSpec((1, 128), lambda i: (0, 0)),
                   pl.BlockSpec((1, 128), lambda i: (0, 0))],
        out_shape=[jax.ShapeDtypeStruct((N_NODES, HID_CH), jnp.float32),
                   jax.ShapeDtypeStruct((1, 128), jnp.float32),
                   jax.ShapeDtypeStruct((1, 128), jnp.float32)],
    )(A1, M1)


def _norm_prep_body(hp_ref, s_ref, q_ref, lnw_ref, lnb_ref, pa_ref,
                    w_ref, b_ref, a_ref, bt_ref):
    n_elem = float(N_NODES * HID_CH)
    mean = jnp.max(s_ref[...]) / n_elem
    var = jnp.max(q_ref[...]) / n_elem - mean * mean
    inv = lax.rsqrt(var + _EPS)
    h = (hp_ref[...] - mean) * inv * lnw_ref[...] + lnb_ref[...]
    pa = jnp.max(pa_ref[...])
    h = jnp.where(h >= 0, h, pa * h)
    wfull = w_ref[...]
    wt = wfull[:HID_CH]
    wb = wfull[HID_CH:]
    a_ref[...] = _mm(h, wt - wb) + b_ref[...]
    bt_ref[...] = _mm(h, wb).astype(jnp.bfloat16)


def _tc_norm_prep(hp, s, q, ln_w, ln_b, prelu_a, W2, b2):
    return pl.pallas_call(
        _norm_prep_body,
        grid=(NRB,),
        in_specs=[pl.BlockSpec((RB, HID_CH), lambda i: (i, 0)),
                  pl.BlockSpec((1, 128), lambda i: (0, 0)),
                  pl.BlockSpec((1, 128), lambda i: (0, 0)),
                  pl.BlockSpec((1, HID_CH), lambda i: (0, 0)),
                  pl.BlockSpec((1, HID_CH), lambda i: (0, 0)),
                  pl.BlockSpec((1, 1), lambda i: (0, 0)),
                  pl.BlockSpec((2 * HID_CH, OUT_CH), lambda i: (0, 0)),
                  pl.BlockSpec((1, OUT_CH), lambda i: (0, 0))],
        out_specs=[pl.BlockSpec((RB, OUT_CH), lambda i: (i, 0)),
                   pl.BlockSpec((RB, OUT_CH), lambda i: (i, 0))],
        out_shape=[jax.ShapeDtypeStruct((N_NODES, OUT_CH), jnp.float32),
                   jax.ShapeDtypeStruct((N_NODES, OUT_CH), jnp.bfloat16)],
    )(hp, s, q, ln_w.reshape(1, HID_CH), ln_b.reshape(1, HID_CH),
      prelu_a.reshape(1, 1), W2, b2.reshape(1, OUT_CH))


def _final_body(a_ref, m_ref, o_ref):
    mv = m_ref[...].astype(jnp.float32)
    o_ref[...] = jnp.where(jnp.isfinite(mv), a_ref[...] + mv, 0.0)


def _tc_final(A2, M2):
    return pl.pallas_call(
        _final_body,
        grid=(NRB,),
        in_specs=[pl.BlockSpec((RB, OUT_CH), lambda i: (i, 0)),
                  pl.BlockSpec((RB, OUT_CH), lambda i: (i, 0))],
        out_specs=pl.BlockSpec((RB, OUT_CH), lambda i: (i, 0)),
        out_shape=jax.ShapeDtypeStruct((N_NODES, OUT_CH), jnp.float32),
    )(A2, M2)


def kernel(x, edge_index, W1, b1, W2, b2, ln_w, ln_b, prelu_a):
    ei = edge_index.astype(jnp.int32)
    src = ei[0]
    dst = ei[1]
    pairs, counts = _sc_bin(src, dst)
    A1, B1 = _tc_prep1(x, W1, b1)
    M1 = _sc_segmax(B1, pairs, counts)[:N_NODES]
    hp, s, q = _tc_stats(A1, M1)
    A2, B2 = _tc_norm_prep(hp, s, q, ln_w, ln_b, prelu_a, W2, b2)
    M2 = _sc_segmax(B2, pairs, counts)[:N_NODES]
    return _tc_final(A2, M2)
